# baseline (speedup 1.0000x reference)
import functools

import jax
import jax.numpy as jnp
from jax import lax
from jax.experimental import pallas as pl
from jax.experimental.pallas import tpu as pltpu
from jax.experimental.pallas import tpu_sc as plsc

EMB_DIM = 16
BATCH = 16384
NUM_ROWS = 1000000
FLAT = NUM_ROWS * EMB_DIM
NUM_CORES = 2
NUM_SUBCORES = 16
NUM_WORKERS = NUM_CORES * NUM_SUBCORES
BPW = BATCH // NUM_WORKERS
CHUNK = 128
NCHUNK = BPW // CHUNK
LANES = 16
NVEC = BPW // LANES
VALS = EMB_DIM * BPW


def _body(iu_hbm, ii_hbm, ut_hbm, it_hbm, bias_hbm, out_hbm,
          idx_u, idx_i, off_u, off_i, vals_u, vals_i, out_v,
          sem_u, sem_i, sem_b):
    wid = lax.axis_index("s") * NUM_CORES + lax.axis_index("c")
    base = wid * BPW

    pltpu.sync_copy(iu_hbm.at[pl.ds(base, BPW)], idx_u)
    pltpu.sync_copy(ii_hbm.at[pl.ds(base, BPW)], idx_i)

    bias_copies = [
        pltpu.async_copy(bias_hbm.at[idx_i.at[pl.ds(c * CHUNK, CHUNK)]],
                         out_v.at[pl.ds(c * CHUNK, CHUNK)], sem_b)
        for c in range(NCHUNK)
    ]

    def expand_body(j, carry):
        sl = pl.ds(j * LANES, LANES)
        iu = idx_u[sl]
        ii = idx_i[sl]
        for d in range(EMB_DIM):
            dsl = pl.ds(d * BPW + j * LANES, LANES)
            off_u[dsl] = iu + d * NUM_ROWS
            off_i[dsl] = ii + d * NUM_ROWS
        return carry

    lax.fori_loop(0, NVEC, expand_body, 0)

    copies = []
    for d in range(EMB_DIM):
        for c in range(NCHUNK):
            sl = pl.ds(d * BPW + c * CHUNK, CHUNK)
            copies.append(pltpu.async_copy(
                ut_hbm.at[off_u.at[sl]], vals_u.at[sl], sem_u))
            copies.append(pltpu.async_copy(
                it_hbm.at[off_i.at[sl]], vals_i.at[sl], sem_i))
    del copies
    pltpu.make_async_copy(ut_hbm.at[pl.ds(0, VALS)], vals_u, sem_u).wait()
    pltpu.make_async_copy(it_hbm.at[pl.ds(0, VALS)], vals_i, sem_i).wait()
    for cb in bias_copies:
        cb.wait()

    def group_body(g, carry):
        nsl = pl.ds(g * LANES, LANES)
        acc = out_v[nsl]
        for d in range(EMB_DIM):
            dsl = pl.ds(d * BPW + g * LANES, LANES)
            acc = acc + vals_u[dsl] * vals_i[dsl]
        out_v[nsl] = acc
        return carry

    lax.fori_loop(0, NVEC, group_body, 0)

    pltpu.sync_copy(out_v, out_hbm.at[pl.ds(base, BPW)])


@jax.jit
def _run(input_user, input_item, u_flat, i_flat, d_item_bias):
    mesh = plsc.VectorSubcoreMesh(
        core_axis_name="c", subcore_axis_name="s",
        num_cores=NUM_CORES, num_subcores=NUM_SUBCORES)
    f = pl.kernel(
        _body,
        out_type=jax.ShapeDtypeStruct((BATCH,), jnp.float32),
        mesh=mesh,
        scratch_types=[
            pltpu.VMEM((BPW,), jnp.int32),
            pltpu.VMEM((BPW,), jnp.int32),
            pltpu.VMEM((VALS,), jnp.int32),
            pltpu.VMEM((VALS,), jnp.int32),
            pltpu.VMEM((VALS,), jnp.float32),
            pltpu.VMEM((VALS,), jnp.float32),
            pltpu.VMEM((BPW,), jnp.float32),
            pltpu.SemaphoreType.DMA,
            pltpu.SemaphoreType.DMA,
            pltpu.SemaphoreType.DMA,
        ],
        compiler_params=pltpu.CompilerParams(
            needs_layout_passes=False, use_tc_tiling_on_sc=False),
    )
    return f(input_user, input_item, u_flat, i_flat, d_item_bias)


def kernel(input_user, input_item, u_table, i_table, d_item_bias):
    return _run(input_user.astype(jnp.int32), input_item.astype(jnp.int32),
                u_table.T.reshape(FLAT), i_table.T.reshape(FLAT),
                d_item_bias)


# speedup vs baseline: 18.2172x; 18.2172x over previous
import functools

import jax
import jax.numpy as jnp
from jax import lax
from jax.experimental import pallas as pl
from jax.experimental.pallas import tpu as pltpu
from jax.experimental.pallas import tpu_sc as plsc

EMB_DIM = 16
BATCH = 16384
NUM_ROWS = 1000000
ROWPITCH = 1000064
FLAT = ROWPITCH * EMB_DIM
NUM_CORES = 2
NUM_SUBCORES = 16
NUM_WORKERS = NUM_CORES * NUM_SUBCORES
BPW = BATCH // NUM_WORKERS
CHUNK = 128
NCHUNK = BPW // CHUNK
LANES = 16
NVEC = BPW // LANES
VALS = EMB_DIM * BPW

WIN = 1024
NFULL = NUM_ROWS // WIN
TAIL = NFULL * WIN
NITER = (NFULL + NUM_WORKERS - 1) // NUM_WORKERS
NBUF = 4


def _relayout_table(src, dst, stg, sem_r, sem_w, wid):
    def window(w):
        return src.at[:, pl.ds(pl.multiple_of(w * WIN, WIN), WIN)]

    for p in range(NBUF):
        @pl.when(wid + 32 * p < NFULL)
        def _(p=p):
            pltpu.async_copy(window(wid + 32 * p), stg.at[p], sem_r.at[p])

    def body(n, carry):
        w = wid + 32 * n
        b = lax.rem(n, NBUF)

        @pl.when(w < NFULL)
        def _():
            pltpu.make_async_copy(window(w), stg.at[b], sem_r.at[b]).wait()
            for d in range(EMB_DIM):
                pltpu.async_copy(
                    stg.at[b].at[d],
                    dst.at[pl.ds(d * ROWPITCH + w * WIN, WIN)], sem_w)
            wn = w + 32 * NBUF

            @pl.when(wn < NFULL)
            def _():
                pltpu.async_copy(window(wn), stg.at[b], sem_r.at[b])

            pltpu.make_async_copy(window(w), stg.at[b], sem_w).wait()
        return carry

    lax.fori_loop(0, NITER, body, 0)

    ntail = (ROWPITCH - TAIL) // 128

    @pl.when(wid == 0)
    def _():
        for t in range(ntail):
            base = TAIL + t * 128
            tail_src = src.at[:, pl.ds(pl.multiple_of(base, 128), 128)]
            tail_stg = stg.at[0].at[:, pl.ds(0, 128)]
            pltpu.async_copy(tail_src, tail_stg, sem_r.at[0])
            pltpu.make_async_copy(tail_src, tail_stg, sem_r.at[0]).wait()
            for d in range(EMB_DIM):
                pltpu.async_copy(stg.at[0].at[d].at[pl.ds(0, 128)],
                                 dst.at[pl.ds(d * ROWPITCH + base, 128)],
                                 sem_w)
            pltpu.make_async_copy(tail_src, tail_stg, sem_w).wait()


def _conv_body(ut_hbm, it_hbm, uf_hbm, if_hbm, stg, sem_r, sem_w):
    wid = lax.axis_index("s") * NUM_CORES + lax.axis_index("c")
    _relayout_table(ut_hbm, uf_hbm, stg, sem_r, sem_w, wid)
    _relayout_table(it_hbm, if_hbm, stg, sem_r, sem_w, wid)


def _body(iu_hbm, ii_hbm, ut_hbm, it_hbm, bias_hbm, out_hbm,
          idx_u, idx_i, off_u, off_i, vals_u, vals_i, out_v,
          sem_u, sem_i, sem_b):
    wid = lax.axis_index("s") * NUM_CORES + lax.axis_index("c")
    base = wid * BPW

    pltpu.sync_copy(iu_hbm.at[pl.ds(base, BPW)], idx_u)
    pltpu.sync_copy(ii_hbm.at[pl.ds(base, BPW)], idx_i)

    bias_copies = [
        pltpu.async_copy(bias_hbm.at[idx_i.at[pl.ds(c * CHUNK, CHUNK)]],
                         out_v.at[pl.ds(c * CHUNK, CHUNK)], sem_b)
        for c in range(NCHUNK)
    ]

    def expand_body(j, carry):
        sl = pl.ds(j * LANES, LANES)
        iu = idx_u[sl]
        ii = idx_i[sl]
        for d in range(EMB_DIM):
            dsl = pl.ds(d * BPW + j * LANES, LANES)
            off_u[dsl] = iu + d * ROWPITCH
            off_i[dsl] = ii + d * ROWPITCH
        return carry

    lax.fori_loop(0, NVEC, expand_body, 0)

    for d in range(EMB_DIM):
        for c in range(NCHUNK):
            sl = pl.ds(d * BPW + c * CHUNK, CHUNK)
            pltpu.async_copy(ut_hbm.at[off_u.at[sl]], vals_u.at[sl], sem_u)
            pltpu.async_copy(it_hbm.at[off_i.at[sl]], vals_i.at[sl], sem_i)
    pltpu.make_async_copy(ut_hbm.at[pl.ds(0, VALS)], vals_u, sem_u).wait()
    pltpu.make_async_copy(it_hbm.at[pl.ds(0, VALS)], vals_i, sem_i).wait()
    for cb in bias_copies:
        cb.wait()

    def group_body(g, carry):
        nsl = pl.ds(g * LANES, LANES)
        acc = out_v[nsl]
        for d in range(EMB_DIM):
            dsl = pl.ds(d * BPW + g * LANES, LANES)
            acc = acc + vals_u[dsl] * vals_i[dsl]
        out_v[nsl] = acc
        return carry

    lax.fori_loop(0, NVEC, group_body, 0)

    pltpu.sync_copy(out_v, out_hbm.at[pl.ds(base, BPW)])


@jax.jit
def _run(input_user, input_item, u_table_t, i_table_t, d_item_bias):
    mesh = plsc.VectorSubcoreMesh(
        core_axis_name="c", subcore_axis_name="s",
        num_cores=NUM_CORES, num_subcores=NUM_SUBCORES)
    conv = pl.kernel(
        _conv_body,
        out_type=(jax.ShapeDtypeStruct((FLAT,), jnp.float32),
                  jax.ShapeDtypeStruct((FLAT,), jnp.float32)),
        mesh=mesh,
        scratch_types=[
            pltpu.VMEM((NBUF, EMB_DIM, WIN), jnp.float32),
            pltpu.SemaphoreType.DMA((NBUF,)),
            pltpu.SemaphoreType.DMA,
        ],
        compiler_params=pltpu.CompilerParams(
            needs_layout_passes=False, use_tc_tiling_on_sc=True,
            disable_bounds_checks=True),
    )
    u_flat, i_flat = conv(u_table_t, i_table_t)

    f = pl.kernel(
        _body,
        out_type=jax.ShapeDtypeStruct((BATCH,), jnp.float32),
        mesh=mesh,
        scratch_types=[
            pltpu.VMEM((BPW,), jnp.int32),
            pltpu.VMEM((BPW,), jnp.int32),
            pltpu.VMEM((VALS,), jnp.int32),
            pltpu.VMEM((VALS,), jnp.int32),
            pltpu.VMEM((VALS,), jnp.float32),
            pltpu.VMEM((VALS,), jnp.float32),
            pltpu.VMEM((BPW,), jnp.float32),
            pltpu.SemaphoreType.DMA,
            pltpu.SemaphoreType.DMA,
            pltpu.SemaphoreType.DMA,
        ],
        compiler_params=pltpu.CompilerParams(
            needs_layout_passes=False, use_tc_tiling_on_sc=False),
    )
    return f(input_user, input_item, u_flat, i_flat, d_item_bias)


def kernel(input_user, input_item, u_table, i_table, d_item_bias):
    return _run(input_user.astype(jnp.int32), input_item.astype(jnp.int32),
                u_table.T, i_table.T, d_item_bias)


# speedup vs baseline: 19.1431x; 1.0508x over previous
import functools

import jax
import jax.numpy as jnp
from jax import lax
from jax.experimental import pallas as pl
from jax.experimental.pallas import tpu as pltpu
from jax.experimental.pallas import tpu_sc as plsc

EMB_DIM = 16
BATCH = 16384
NUM_ROWS = 1000000
ROWPITCH = 1000064
FLAT = ROWPITCH * EMB_DIM
NUM_CORES = 2
NUM_SUBCORES = 16
NUM_WORKERS = NUM_CORES * NUM_SUBCORES
BPW = BATCH // NUM_WORKERS
CHUNK = 128
NCHUNK = BPW // CHUNK
LANES = 16
NVEC = BPW // LANES
VALS = EMB_DIM * BPW

WIN = 3968
NFULL = NUM_ROWS // WIN
TAIL = NFULL * WIN
NITER = (NFULL + NUM_WORKERS - 1) // NUM_WORKERS
NBUF = 2


def _relayout_table(src, dst, stg, sem_r, sem_w, wid):
    def window(w):
        return src.at[:, pl.ds(pl.multiple_of(w * WIN, WIN), WIN)]

    for p in range(NBUF):
        @pl.when(wid + NUM_WORKERS * p < NFULL)
        def _(p=p):
            pltpu.async_copy(window(wid + NUM_WORKERS * p), stg.at[p],
                             sem_r.at[p])

    def body(n, carry):
        w = wid + NUM_WORKERS * n
        b = lax.rem(n, NBUF)

        @pl.when(w < NFULL)
        def _():
            pltpu.make_async_copy(window(w), stg.at[b], sem_r.at[b]).wait()
            for d in range(EMB_DIM):
                pltpu.async_copy(
                    stg.at[b].at[d],
                    dst.at[pl.ds(d * ROWPITCH + w * WIN, WIN)], sem_w.at[b])
            wn = w + NUM_WORKERS * NBUF

            @pl.when(wn < NFULL)
            def _():
                pltpu.async_copy(window(wn), stg.at[b], sem_r.at[b])

            pltpu.make_async_copy(window(w), stg.at[b], sem_w.at[b]).wait()
        return carry

    lax.fori_loop(0, NITER, body, 0)

    ntail = (ROWPITCH - TAIL) // 128

    for t in range(ntail):
        @pl.when(wid == t)
        def _(t=t):
            base = TAIL + t * 128
            tail_src = src.at[:, pl.ds(pl.multiple_of(base, 128), 128)]
            tail_stg = stg.at[0].at[:, pl.ds(0, 128)]
            pltpu.async_copy(tail_src, tail_stg, sem_r.at[0])
            pltpu.make_async_copy(tail_src, tail_stg, sem_r.at[0]).wait()
            for d in range(EMB_DIM):
                pltpu.async_copy(stg.at[0].at[d].at[pl.ds(0, 128)],
                                 dst.at[pl.ds(d * ROWPITCH + base, 128)],
                                 sem_w.at[0])
            pltpu.make_async_copy(tail_src, tail_stg, sem_w.at[0]).wait()


def _conv_body(ut_hbm, it_hbm, uf_hbm, if_hbm, stg, sem_r, sem_w):
    wid = lax.axis_index("s") * NUM_CORES + lax.axis_index("c")
    _relayout_table(ut_hbm, uf_hbm, stg, sem_r, sem_w, wid)
    _relayout_table(it_hbm, if_hbm, stg, sem_r, sem_w, wid)


def _body(iu_hbm, ii_hbm, ut_hbm, it_hbm, bias_hbm, out_hbm,
          idx_u, idx_i, off_u, off_i, vals_u, vals_i, out_v,
          sem_u, sem_i, sem_b):
    wid = lax.axis_index("s") * NUM_CORES + lax.axis_index("c")
    base = wid * BPW

    pltpu.sync_copy(iu_hbm.at[pl.ds(base, BPW)], idx_u)
    pltpu.sync_copy(ii_hbm.at[pl.ds(base, BPW)], idx_i)

    bias_copies = [
        pltpu.async_copy(bias_hbm.at[idx_i.at[pl.ds(c * CHUNK, CHUNK)]],
                         out_v.at[pl.ds(c * CHUNK, CHUNK)], sem_b)
        for c in range(NCHUNK)
    ]

    def expand_body(j, carry):
        sl = pl.ds(j * LANES, LANES)
        iu = idx_u[sl]
        ii = idx_i[sl]
        for d in range(EMB_DIM):
            dsl = pl.ds(d * BPW + j * LANES, LANES)
            off_u[dsl] = iu + d * ROWPITCH
            off_i[dsl] = ii + d * ROWPITCH
        return carry

    lax.fori_loop(0, NVEC, expand_body, 0)

    for d in range(EMB_DIM):
        for c in range(NCHUNK):
            sl = pl.ds(d * BPW + c * CHUNK, CHUNK)
            pltpu.async_copy(ut_hbm.at[off_u.at[sl]], vals_u.at[sl], sem_u)
            pltpu.async_copy(it_hbm.at[off_i.at[sl]], vals_i.at[sl], sem_i)
    pltpu.make_async_copy(ut_hbm.at[pl.ds(0, VALS)], vals_u, sem_u).wait()
    pltpu.make_async_copy(it_hbm.at[pl.ds(0, VALS)], vals_i, sem_i).wait()
    for cb in bias_copies:
        cb.wait()

    def group_body(g, carry):
        nsl = pl.ds(g * LANES, LANES)
        acc = out_v[nsl]
        for d in range(EMB_DIM):
            dsl = pl.ds(d * BPW + g * LANES, LANES)
            acc = acc + vals_u[dsl] * vals_i[dsl]
        out_v[nsl] = acc
        return carry

    lax.fori_loop(0, NVEC, group_body, 0)

    pltpu.sync_copy(out_v, out_hbm.at[pl.ds(base, BPW)])


@jax.jit
def _run(input_user, input_item, u_table_t, i_table_t, d_item_bias):
    mesh = plsc.VectorSubcoreMesh(
        core_axis_name="c", subcore_axis_name="s",
        num_cores=NUM_CORES, num_subcores=NUM_SUBCORES)
    conv = pl.kernel(
        _conv_body,
        out_type=(jax.ShapeDtypeStruct((FLAT,), jnp.float32),
                  jax.ShapeDtypeStruct((FLAT,), jnp.float32)),
        mesh=mesh,
        scratch_types=[
            pltpu.VMEM((NBUF, EMB_DIM, WIN), jnp.float32),
            pltpu.SemaphoreType.DMA((NBUF,)),
            pltpu.SemaphoreType.DMA((NBUF,)),
        ],
        compiler_params=pltpu.CompilerParams(
            needs_layout_passes=False, use_tc_tiling_on_sc=True,
            disable_bounds_checks=True),
    )
    u_flat, i_flat = conv(u_table_t, i_table_t)

    f = pl.kernel(
        _body,
        out_type=jax.ShapeDtypeStruct((BATCH,), jnp.float32),
        mesh=mesh,
        scratch_types=[
            pltpu.VMEM((BPW,), jnp.int32),
            pltpu.VMEM((BPW,), jnp.int32),
            pltpu.VMEM((VALS,), jnp.int32),
            pltpu.VMEM((VALS,), jnp.int32),
            pltpu.VMEM((VALS,), jnp.float32),
            pltpu.VMEM((VALS,), jnp.float32),
            pltpu.VMEM((BPW,), jnp.float32),
            pltpu.SemaphoreType.DMA,
            pltpu.SemaphoreType.DMA,
            pltpu.SemaphoreType.DMA,
        ],
        compiler_params=pltpu.CompilerParams(
            needs_layout_passes=False, use_tc_tiling_on_sc=False),
    )
    return f(input_user, input_item, u_flat, i_flat, d_item_bias)


def kernel(input_user, input_item, u_table, i_table, d_item_bias):
    return _run(input_user.astype(jnp.int32), input_item.astype(jnp.int32),
                u_table.T, i_table.T, d_item_bias)
